# Initial kernel scaffold; baseline (speedup 1.0000x reference)
import jax
import jax.numpy as jnp
from jax.experimental import pallas as pl


def kernel(asu_id, hkl, miller_id, dHKL, seen):
    raise NotImplementedError("write your pallas kernel here")


# speedup vs baseline: 22.8591x; 22.8591x over previous
import jax
import jax.numpy as jnp
from jax import lax
from jax.experimental import pallas as pl
from jax.experimental.pallas import tpu as pltpu
from jax.experimental.pallas import tpu_sc as plsc

N_ASU = 2
GRID = 121
G2 = GRID * GRID
G3 = GRID * G2
ASU_SIZE = 2 * 524288
B = 1048576

NC, NS, L = 2, 16, 16
NW = NC * NS
BPW = B // NW
CH = 2048
NCH = BPW // CH
CROWS = CH // 128
SEEN_SL = ASU_SIZE // NS


def _sc_body(aid_hbm, h_hbm, k_hbm, l_hbm, miller_hbm, seen_hbm,
             out_hbm, seen0_hbm, seen1_hbm,
             asu_v, h_v, k_v, l_v, idx_v, out_v, ones_v, seen_sp, sem):
    c = lax.axis_index("c")
    s = lax.axis_index("s")
    wid = c * NS + s

    for i in range(128 // L):
        ones_v[pl.ds(i * L, L)] = jnp.ones((L,), jnp.int32)

    pltpu.sync_copy(seen_hbm.at[pl.ds(s * SEEN_SL, SEEN_SL)],
                    seen_sp.at[pl.ds(s * SEEN_SL, SEEN_SL)])
    plsc.subcore_barrier()

    @pl.loop(0, NCH)
    def _chunk(t):
        base = wid * BPW + t * CH
        pltpu.sync_copy(aid_hbm.at[pl.ds(base, CH)], asu_v)
        pltpu.sync_copy(h_hbm.at[pl.ds(base, CH)], h_v)
        pltpu.sync_copy(k_hbm.at[pl.ds(base, CH)], k_v)
        pltpu.sync_copy(l_hbm.at[pl.ds(base, CH)], l_v)

        @pl.loop(0, CH // L)
        def _compute(i):
            sl = pl.ds(i * L, L)
            idx_v[sl] = (asu_v[sl] * G3 + h_v[sl] * G2
                         + k_v[sl] * GRID + l_v[sl])

        gathers = [
            pltpu.async_copy(miller_hbm.at[idx_v.at[pl.ds(j * 128, 128)]],
                             out_v.at[j], sem)
            for j in range(CROWS)
        ]
        for g in gathers:
            g.wait()

        for j in range(CROWS):
            pltpu.sync_copy(ones_v, seen_sp.at[out_v.at[j]], add=True)

        row = wid * (BPW // 128) + t * CROWS
        pltpu.sync_copy(out_v, out_hbm.at[pl.ds(row, CROWS)])

    plsc.subcore_barrier()
    sl = pl.ds(s * SEEN_SL, SEEN_SL)

    @pl.when(c == 0)
    def _():
        pltpu.sync_copy(seen_sp.at[sl], seen0_hbm.at[sl])

    @pl.when(c == 1)
    def _():
        pltpu.sync_copy(seen_sp.at[sl], seen1_hbm.at[sl])


def _sc_gather_scatter(aid, h, k, l, miller, seen_i32):
    mesh = plsc.VectorSubcoreMesh(core_axis_name="c", subcore_axis_name="s")
    f = pl.kernel(
        _sc_body,
        out_type=(jax.ShapeDtypeStruct((B // 128, 128), jnp.int32),
                  jax.ShapeDtypeStruct((ASU_SIZE,), jnp.int32),
                  jax.ShapeDtypeStruct((ASU_SIZE,), jnp.int32)),
        mesh=mesh,
        scratch_types=[
            pltpu.VMEM((CH,), jnp.int32),
            pltpu.VMEM((CH,), jnp.int32),
            pltpu.VMEM((CH,), jnp.int32),
            pltpu.VMEM((CH,), jnp.int32),
            pltpu.VMEM((CH,), jnp.int32),
            pltpu.VMEM((CROWS, 128), jnp.int32),
            pltpu.VMEM((128,), jnp.int32),
            pltpu.VMEM_SHARED((ASU_SIZE,), jnp.int32),
            pltpu.SemaphoreType.DMA,
        ],
    )
    return f(aid, h, k, l, miller, seen_i32)


def _combine_body(s0_ref, s1_ref, o_ref):
    o_ref[...] = (s0_ref[...] | s1_ref[...]) != 0


def _combine(seen0, seen1):
    nrows = ASU_SIZE // 128
    blk = 1024
    spec = pl.BlockSpec((blk, 128), lambda i: (i, 0))
    return pl.pallas_call(
        _combine_body,
        grid=(nrows // blk,),
        in_specs=[spec, spec],
        out_specs=spec,
        out_shape=jax.ShapeDtypeStruct((nrows, 128), jnp.bool_),
    )(seen0.reshape(nrows, 128), seen1.reshape(nrows, 128))


def kernel(asu_id, hkl, miller_id, dHKL, seen):
    del dHKL
    aid = asu_id.reshape(B)
    hklt = jnp.transpose(hkl)
    miller = miller_id.reshape(N_ASU * G3)
    out2d, seen0, seen1 = _sc_gather_scatter(
        aid, hklt[0], hklt[1], hklt[2], miller, seen.astype(jnp.int32))
    seen_new = _combine(seen0, seen1).reshape(ASU_SIZE)
    return out2d.reshape(B), seen_new


# speedup vs baseline: 27.6501x; 1.2096x over previous
import jax
import jax.numpy as jnp
from jax import lax
from jax.experimental import pallas as pl
from jax.experimental.pallas import tpu as pltpu
from jax.experimental.pallas import tpu_sc as plsc

N_ASU = 2
GRID = 121
G2 = GRID * GRID
G3 = GRID * G2
ASU_SIZE = 2 * 524288
B = 1048576

NC, NS, L = 2, 16, 16
NW = NC * NS
BPW = B // NW
CH = 2048
NCH = BPW // CH
CROWS = CH // 128
SEEN_SL = ASU_SIZE // NS


def _sc_body(aid_hbm, h_hbm, k_hbm, l_hbm, miller_hbm, seen_hbm,
             out_hbm, seen0_hbm, seen1_hbm,
             asu_v, h_v, k_v, l_v, idx_v, out_v, ones_v, seen_sp, sem):
    c = lax.axis_index("c")
    s = lax.axis_index("s")
    wid = c * NS + s

    @pl.loop(0, CH // L)
    def _ones(i):
        ones_v[pl.ds(i * L, L)] = jnp.ones((L,), jnp.int32)

    pltpu.sync_copy(seen_hbm.at[pl.ds(s * SEEN_SL, SEEN_SL)],
                    seen_sp.at[pl.ds(s * SEEN_SL, SEEN_SL)])
    plsc.subcore_barrier()

    @pl.loop(0, NCH)
    def _chunk(t):
        base = wid * BPW + t * CH
        stages = [
            pltpu.async_copy(aid_hbm.at[pl.ds(base, CH)], asu_v, sem),
            pltpu.async_copy(h_hbm.at[pl.ds(base, CH)], h_v, sem),
            pltpu.async_copy(k_hbm.at[pl.ds(base, CH)], k_v, sem),
            pltpu.async_copy(l_hbm.at[pl.ds(base, CH)], l_v, sem),
        ]
        for st in stages:
            st.wait()

        @pl.loop(0, CH // L)
        def _compute(i):
            sl = pl.ds(i * L, L)
            idx_v[sl] = (asu_v[sl] * G3 + h_v[sl] * G2
                         + k_v[sl] * GRID + l_v[sl])

        pltpu.async_copy(miller_hbm.at[idx_v], out_v, sem).wait()
        pltpu.sync_copy(ones_v, seen_sp.at[out_v], add=True)
        pltpu.sync_copy(out_v, out_hbm.at[pl.ds(base, CH)])

    plsc.subcore_barrier()
    sl = pl.ds(s * SEEN_SL, SEEN_SL)

    @pl.when(c == 0)
    def _():
        pltpu.sync_copy(seen_sp.at[sl], seen0_hbm.at[sl])

    @pl.when(c == 1)
    def _():
        pltpu.sync_copy(seen_sp.at[sl], seen1_hbm.at[sl])


def _sc_gather_scatter(aid, h, k, l, miller, seen_i32):
    mesh = plsc.VectorSubcoreMesh(core_axis_name="c", subcore_axis_name="s")
    f = pl.kernel(
        _sc_body,
        out_type=(jax.ShapeDtypeStruct((B,), jnp.int32),
                  jax.ShapeDtypeStruct((ASU_SIZE,), jnp.int32),
                  jax.ShapeDtypeStruct((ASU_SIZE,), jnp.int32)),
        mesh=mesh,
        scratch_types=[
            pltpu.VMEM((CH,), jnp.int32),
            pltpu.VMEM((CH,), jnp.int32),
            pltpu.VMEM((CH,), jnp.int32),
            pltpu.VMEM((CH,), jnp.int32),
            pltpu.VMEM((CH,), jnp.int32),
            pltpu.VMEM((CH,), jnp.int32),
            pltpu.VMEM((CH,), jnp.int32),
            pltpu.VMEM_SHARED((ASU_SIZE,), jnp.int32),
            pltpu.SemaphoreType.DMA,
        ],
    )
    return f(aid, h, k, l, miller, seen_i32)


def _combine_body(s0_ref, s1_ref, o_ref):
    o_ref[...] = (s0_ref[...] | s1_ref[...]) != 0


def _combine(seen0, seen1):
    nrows = ASU_SIZE // 128
    blk = 1024
    spec = pl.BlockSpec((blk, 128), lambda i: (i, 0))
    return pl.pallas_call(
        _combine_body,
        grid=(nrows // blk,),
        in_specs=[spec, spec],
        out_specs=spec,
        out_shape=jax.ShapeDtypeStruct((nrows, 128), jnp.bool_),
    )(seen0.reshape(nrows, 128), seen1.reshape(nrows, 128))


def kernel(asu_id, hkl, miller_id, dHKL, seen):
    del dHKL
    aid = asu_id.reshape(B)
    hklt = jnp.transpose(hkl)
    miller = miller_id.reshape(N_ASU * G3)
    out, seen0, seen1 = _sc_gather_scatter(
        aid, hklt[0], hklt[1], hklt[2], miller, seen.astype(jnp.int32))
    seen_new = _combine(seen0, seen1).reshape(ASU_SIZE)
    return out, seen_new
